# baseline (speedup 1.0000x reference)
import functools

import jax
import jax.numpy as jnp
from jax import lax
from jax.experimental import pallas as pl
from jax.experimental.pallas import tpu as pltpu
from jax.experimental.pallas import tpu_sc as plsc

EPS = 1e-30
LANES = 16
NSPLIT = 2


def _log_vec(v):
    bits = plsc.bitcast(v, jnp.int32)
    e = lax.shift_right_logical(bits, jnp.int32(23)) - jnp.int32(127)
    m = plsc.bitcast(
        lax.bitwise_or(lax.bitwise_and(bits, jnp.int32(0x007FFFFF)), jnp.int32(0x3F800000)),
        jnp.float32,
    )
    ef = e.astype(jnp.float32)
    cond = m > jnp.float32(1.4142135)
    m = jnp.where(cond, m * jnp.float32(0.5), m)
    ef = jnp.where(cond, ef + jnp.float32(1.0), ef)
    s = (m - jnp.float32(1.0)) / (m + jnp.float32(1.0))
    z = s * s
    logm = s * (
        jnp.float32(2.0)
        + z * (jnp.float32(0.6666667) + z * (jnp.float32(0.4) + z * jnp.float32(0.28571429)))
    )
    return ef * jnp.float32(0.69314718) + logm


def _sc_info():
    try:
        info = plsc.get_sparse_core_info()
        return info.num_cores, info.num_subcores
    except Exception:
        return 2, 16


def kernel(batch_n_nodes, prob, num_nodes):
    del num_nodes
    batch = batch_n_nodes.shape[0]
    nbuckets = prob.shape[0]
    num_cores, num_subcores = 2, 16
    nw = num_cores * num_subcores
    assert batch % (8 * nw) == 0
    chunk = batch // nw
    mesh = plsc.VectorSubcoreMesh(
        core_axis_name="c", subcore_axis_name="s", num_cores=num_cores, num_subcores=num_subcores
    )

    @functools.partial(
        pl.kernel,
        mesh=mesh,
        out_type=jax.ShapeDtypeStruct((batch,), jnp.float32),
        compiler_params=pltpu.CompilerParams(
            needs_layout_passes=False,
            skip_device_barrier=True,
            disable_bounds_checks=True,
            disable_semaphore_checks=True,
        ),
        scratch_types=[
            pltpu.VMEM((nbuckets,), jnp.float32),
            pltpu.VMEM((chunk,), jnp.int32),
            pltpu.VMEM((chunk,), jnp.float32),
            pltpu.SemaphoreType.DMA,
            [pltpu.SemaphoreType.DMA] * NSPLIT,
        ],
    )
    def run(idx_hbm, prob_hbm, out_hbm, table_v, idx_v, out_v, sem_p, sems_i):
        sem_o = sem_p
        wid = lax.axis_index("s") * num_cores + lax.axis_index("c")
        base = wid * chunk
        piece = chunk // NSPLIT
        cp_p = pltpu.async_copy(prob_hbm, table_v, sem_p)
        cps = [
            pltpu.async_copy(
                idx_hbm.at[pl.ds(base + q * piece, piece)],
                idx_v.at[pl.ds(q * piece, piece)],
                sems_i[q],
            )
            for q in range(NSPLIT)
        ]
        cp_p.wait()
        for i in range(nbuckets // LANES):
            sl = pl.ds(i * LANES, LANES)
            table_v[sl] = _log_vec(table_v[sl] + jnp.float32(EPS))

        def gather_range(lo, hi):
            @plsc.parallel_loop(lo, hi, step=LANES, unroll=8)
            def _(j):
                sl = pl.ds(j, LANES)
                out_v[sl] = plsc.load_gather(table_v, [idx_v[sl]])

        sts = []
        for q in range(NSPLIT):
            cps[q].wait()
            gather_range(q * piece, (q + 1) * piece)
            sts.append(
                pltpu.async_copy(
                    out_v.at[pl.ds(q * piece, piece)],
                    out_hbm.at[pl.ds(base + q * piece, piece)],
                    sem_o,
                )
            )
        for st in sts:
            st.wait()

    return run(batch_n_nodes, prob)


# speedup vs baseline: 1.0755x; 1.0755x over previous
import functools

import jax
import jax.numpy as jnp
from jax import lax
from jax.experimental import pallas as pl
from jax.experimental.pallas import tpu as pltpu
from jax.experimental.pallas import tpu_sc as plsc

EPS = 1e-30
LANES = 16
NSPLIT = 2


def _log_vec(v):
    bits = plsc.bitcast(v, jnp.int32)
    e = lax.shift_right_logical(bits, jnp.int32(23)) - jnp.int32(127)
    m = plsc.bitcast(
        lax.bitwise_or(lax.bitwise_and(bits, jnp.int32(0x007FFFFF)), jnp.int32(0x3F800000)),
        jnp.float32,
    )
    ef = e.astype(jnp.float32)
    cond = m > jnp.float32(1.4142135)
    m = jnp.where(cond, m * jnp.float32(0.5), m)
    ef = jnp.where(cond, ef + jnp.float32(1.0), ef)
    s = (m - jnp.float32(1.0)) / (m + jnp.float32(1.0))
    z = s * s
    logm = s * (
        jnp.float32(2.0)
        + z * (jnp.float32(0.6666667) + z * (jnp.float32(0.4) + z * jnp.float32(0.28571429)))
    )
    return ef * jnp.float32(0.69314718) + logm


def kernel(batch_n_nodes, prob, num_nodes):
    del num_nodes
    batch = batch_n_nodes.shape[0]
    nbuckets = prob.shape[0]
    num_cores, num_subcores = 1, 16
    nw = num_cores * num_subcores
    assert batch % (8 * nw) == 0
    chunk = batch // nw
    mesh = plsc.VectorSubcoreMesh(
        core_axis_name="c", subcore_axis_name="s", num_cores=num_cores, num_subcores=num_subcores
    )

    @functools.partial(
        pl.kernel,
        mesh=mesh,
        out_type=jax.ShapeDtypeStruct((batch,), jnp.float32),
        compiler_params=pltpu.CompilerParams(
            needs_layout_passes=False,
            skip_device_barrier=True,
            disable_bounds_checks=True,
            disable_semaphore_checks=True,
        ),
        scratch_types=[
            pltpu.VMEM((nbuckets,), jnp.float32),
            pltpu.VMEM((chunk,), jnp.int32),
            pltpu.VMEM((chunk,), jnp.float32),
            pltpu.SemaphoreType.DMA,
            [pltpu.SemaphoreType.DMA] * NSPLIT,
        ],
    )
    def run(idx_hbm, prob_hbm, out_hbm, table_v, idx_v, out_v, sem_p, sems_i):
        sem_o = sem_p
        wid = lax.axis_index("s") * num_cores + lax.axis_index("c")
        base = wid * chunk
        piece = chunk // NSPLIT
        cp_p = pltpu.async_copy(prob_hbm, table_v, sem_p)
        cps = [
            pltpu.async_copy(
                idx_hbm.at[pl.ds(base + q * piece, piece)],
                idx_v.at[pl.ds(q * piece, piece)],
                sems_i[q],
            )
            for q in range(NSPLIT)
        ]
        cp_p.wait()
        for i in range(nbuckets // LANES):
            sl = pl.ds(i * LANES, LANES)
            table_v[sl] = _log_vec(table_v[sl] + jnp.float32(EPS))

        def gather_range(lo, hi):
            @plsc.parallel_loop(lo, hi, step=LANES, unroll=8)
            def _(j):
                sl = pl.ds(j, LANES)
                out_v[sl] = plsc.load_gather(table_v, [idx_v[sl]])

        sts = []
        for q in range(NSPLIT):
            cps[q].wait()
            gather_range(q * piece, (q + 1) * piece)
            sts.append(
                pltpu.async_copy(
                    out_v.at[pl.ds(q * piece, piece)],
                    out_hbm.at[pl.ds(base + q * piece, piece)],
                    sem_o,
                )
            )
        for st in sts:
            st.wait()

    return run(batch_n_nodes, prob)
